# Initial kernel scaffold; baseline (speedup 1.0000x reference)
import jax
import jax.numpy as jnp
from jax.experimental import pallas as pl


def kernel(input_ids, U_table, V_table, W, b):
    raise NotImplementedError("write your pallas kernel here")


# speedup vs baseline: 2.8096x; 2.8096x over previous
import functools

import jax
import jax.numpy as jnp
from jax import lax
from jax.experimental import pallas as pl
from jax.experimental.pallas import tpu as pltpu
import jax.experimental.pallas.tpu_sc as plsc

NW = 32
CHUNK = 128
GROUP = 5
NGROUP = 10
NCHUNK = GROUP * NGROUP
P = 16
ALPHA = 4
D = 64


def _sc_gather(ids3, u_table, v_table_i32):
    T = NW * NCHUNK * CHUNK
    mesh = plsc.VectorSubcoreMesh(
        core_axis_name="c", subcore_axis_name="s", num_cores=2, num_subcores=16)

    @functools.partial(
        pl.kernel,
        out_type=(jax.ShapeDtypeStruct((T, P), jnp.float32),
                  jax.ShapeDtypeStruct((T, P * ALPHA // 2), jnp.int32)),
        mesh=mesh,
        compiler_params=pltpu.CompilerParams(use_tc_tiling_on_sc=False),
        scratch_types=[
            pltpu.VMEM((NCHUNK, CHUNK), jnp.int32),
            pltpu.VMEM((GROUP * CHUNK, P), jnp.float32),
            pltpu.VMEM((GROUP * CHUNK, P * ALPHA // 2), jnp.int32),
            pltpu.SemaphoreType.DMA,
            pltpu.SemaphoreType.DMA,
        ],
    )
    def gather_kernel(ids_hbm, u_hbm, v_hbm, u_out, v_out,
                      ids_v, u_v, v_v, su, sv):
        wid = lax.axis_index("s") * 2 + lax.axis_index("c")
        base = wid * (NCHUNK * CHUNK)
        pltpu.sync_copy(ids_hbm.at[wid], ids_v)

        def group_body(g, carry):
            handles = []
            for k in range(GROUP):
                idx = ids_v.at[g * GROUP + k]
                handles.append(pltpu.async_copy(
                    u_hbm.at[idx], u_v.at[pl.ds(k * CHUNK, CHUNK)], su))
                handles.append(pltpu.async_copy(
                    v_hbm.at[idx], v_v.at[pl.ds(k * CHUNK, CHUNK)], sv))
            for h in handles:
                h.wait()
            off = base + g * (GROUP * CHUNK)
            pltpu.sync_copy(u_v, u_out.at[pl.ds(off, GROUP * CHUNK)])
            pltpu.sync_copy(v_v, v_out.at[pl.ds(off, GROUP * CHUNK)])
            return carry

        lax.fori_loop(0, NGROUP, group_body, 0)

    return gather_kernel(ids3, u_table, v_table_i32)


def _tc_project(u_flat, v_bf, wt, b2):
    T = u_flat.shape[0]
    TB = 2048

    def body(u_ref, v_ref, wt_ref, b_ref, x_ref, f_ref, y_ref):
        u = u_ref[...]
        f = jnp.where(u > 0, 1.0, 0.0).astype(jnp.float32)
        f_ref[...] = f
        rows = lax.broadcasted_iota(jnp.int32, (P, P * ALPHA), 0)
        cols = lax.broadcasted_iota(jnp.int32, (P, P * ALPHA), 1)
        expand = (cols // ALPHA == rows).astype(jnp.float32)
        fexp = jnp.dot(f, expand, preferred_element_type=jnp.float32)
        y = v_ref[...].astype(jnp.float32) * fexp
        y_ref[...] = y
        x_ref[...] = jnp.dot(y, wt_ref[...], preferred_element_type=jnp.float32,
                             precision=lax.Precision.HIGHEST) + b_ref[...]

    return pl.pallas_call(
        body,
        grid=(T // TB,),
        in_specs=[pl.BlockSpec((TB, P), lambda i: (i, 0)),
                  pl.BlockSpec((TB, P * ALPHA), lambda i: (i, 0)),
                  pl.BlockSpec((P * ALPHA, D), lambda i: (0, 0)),
                  pl.BlockSpec((1, D), lambda i: (0, 0))],
        out_specs=[pl.BlockSpec((TB, D), lambda i: (i, 0)),
                   pl.BlockSpec((TB, P), lambda i: (i, 0)),
                   pl.BlockSpec((TB, P * ALPHA), lambda i: (i, 0))],
        out_shape=[jax.ShapeDtypeStruct((T, D), jnp.float32),
                   jax.ShapeDtypeStruct((T, P), jnp.float32),
                   jax.ShapeDtypeStruct((T, P * ALPHA), jnp.float32)],
    )(u_flat, v_bf, wt, b2)


def kernel(input_ids, U_table, V_table, W, b):
    B, L = input_ids.shape
    T = B * L
    assert T == NW * NCHUNK * CHUNK
    vocab = U_table.shape[0]
    ids3 = input_ids.reshape(NW, NCHUNK, CHUNK).astype(jnp.int32)
    v_i32 = lax.bitcast_convert_type(
        V_table.reshape(vocab, P * ALPHA // 2, 2), jnp.int32)
    u_flat, v_flat_i32 = _sc_gather(ids3, U_table, v_i32)
    v_bf = lax.bitcast_convert_type(v_flat_i32, jnp.bfloat16).reshape(T, P * ALPHA)
    x_flat, f_flat, y_flat = _tc_project(u_flat, v_bf, W.T, b.reshape(1, D))
    return (x_flat.reshape(B, L, D), u_flat.reshape(B, L, P),
            f_flat.reshape(B, L, P), y_flat.reshape(B, L, P, ALPHA))


# speedup vs baseline: 4.1606x; 1.4808x over previous
import functools

import jax
import jax.numpy as jnp
from jax import lax
from jax.experimental import pallas as pl
from jax.experimental.pallas import tpu as pltpu
import jax.experimental.pallas.tpu_sc as plsc

NW = 32
CHUNK = 128
GROUP = 5
NGROUP = 10
NCHUNK = GROUP * NGROUP
P = 16
ALPHA = 4
D = 64


def _sc_gather(ids3, u_table, v_table2):
    T = NW * NCHUNK * CHUNK
    mesh = plsc.VectorSubcoreMesh(
        core_axis_name="c", subcore_axis_name="s", num_cores=2, num_subcores=16)

    @functools.partial(
        pl.kernel,
        out_type=(jax.ShapeDtypeStruct((T, P), jnp.float32),
                  jax.ShapeDtypeStruct((T, P * ALPHA), jnp.bfloat16)),
        mesh=mesh,
        compiler_params=pltpu.CompilerParams(use_tc_tiling_on_sc=False),
        scratch_types=[
            pltpu.VMEM((NCHUNK, CHUNK), jnp.int32),
            pltpu.VMEM((GROUP * CHUNK, P), jnp.float32),
            pltpu.VMEM((GROUP * CHUNK, P * ALPHA), jnp.bfloat16),
            pltpu.SemaphoreType.DMA,
            pltpu.SemaphoreType.DMA,
        ],
    )
    def gather_kernel(ids_hbm, u_hbm, v_hbm, u_out, v_out,
                      ids_v, u_v, v_v, su, sv):
        wid = lax.axis_index("s") * 2 + lax.axis_index("c")
        base = wid * (NCHUNK * CHUNK)
        pltpu.sync_copy(ids_hbm.at[wid], ids_v)

        def group_body(g, carry):
            handles = []
            for k in range(GROUP):
                idx = ids_v.at[g * GROUP + k]
                handles.append(pltpu.async_copy(
                    u_hbm.at[idx], u_v.at[pl.ds(k * CHUNK, CHUNK)], su))
                handles.append(pltpu.async_copy(
                    v_hbm.at[idx], v_v.at[pl.ds(k * CHUNK, CHUNK)], sv))
            for h in handles:
                h.wait()
            off = base + g * (GROUP * CHUNK)
            pltpu.sync_copy(u_v, u_out.at[pl.ds(off, GROUP * CHUNK)])
            pltpu.sync_copy(v_v, v_out.at[pl.ds(off, GROUP * CHUNK)])
            return carry

        lax.fori_loop(0, NGROUP, group_body, 0)

    return gather_kernel(ids3, u_table, v_table2)


def _tc_project(u_flat, v_bf, wt, b2):
    T = u_flat.shape[0]
    TB = 2048

    def body(u_ref, v_ref, wt_ref, b_ref, x_ref, f_ref, y_ref):
        u = u_ref[...]
        f = jnp.where(u > 0, 1.0, 0.0).astype(jnp.float32)
        f_ref[...] = f
        rows = lax.broadcasted_iota(jnp.int32, (P, P * ALPHA), 0)
        cols = lax.broadcasted_iota(jnp.int32, (P, P * ALPHA), 1)
        expand = (cols // ALPHA == rows).astype(jnp.float32)
        fexp = jnp.dot(f, expand, preferred_element_type=jnp.float32)
        y = v_ref[...].astype(jnp.float32) * fexp
        y_ref[...] = y
        x_ref[...] = jnp.dot(y, wt_ref[...], preferred_element_type=jnp.float32,
                             precision=lax.Precision.HIGHEST) + b_ref[...]

    return pl.pallas_call(
        body,
        grid=(T // TB,),
        in_specs=[pl.BlockSpec((TB, P), lambda i: (i, 0)),
                  pl.BlockSpec((TB, P * ALPHA), lambda i: (i, 0)),
                  pl.BlockSpec((P * ALPHA, D), lambda i: (0, 0)),
                  pl.BlockSpec((1, D), lambda i: (0, 0))],
        out_specs=[pl.BlockSpec((TB, D), lambda i: (i, 0)),
                   pl.BlockSpec((TB, P), lambda i: (i, 0)),
                   pl.BlockSpec((TB, P * ALPHA), lambda i: (i, 0))],
        out_shape=[jax.ShapeDtypeStruct((T, D), jnp.float32),
                   jax.ShapeDtypeStruct((T, P), jnp.float32),
                   jax.ShapeDtypeStruct((T, P * ALPHA), jnp.float32)],
    )(u_flat, v_bf, wt, b2)


def kernel(input_ids, U_table, V_table, W, b):
    B, L = input_ids.shape
    T = B * L
    assert T == NW * NCHUNK * CHUNK
    vocab = U_table.shape[0]
    ids3 = input_ids.reshape(NW, NCHUNK, CHUNK).astype(jnp.int32)
    u_flat, v_bf = _sc_gather(ids3, U_table, V_table.reshape(vocab, P * ALPHA))
    x_flat, f_flat, y_flat = _tc_project(u_flat, v_bf, W.T, b.reshape(1, D))
    return (x_flat.reshape(B, L, D), u_flat.reshape(B, L, P),
            f_flat.reshape(B, L, P), y_flat.reshape(B, L, P, ALPHA))
